# baseline (speedup 1.0000x reference)
import functools

import jax
import jax.numpy as jnp
from jax import lax
from jax.experimental import pallas as pl
from jax.experimental.pallas import tpu as pltpu
from jax.experimental.pallas import tpu_sc as plsc

N = 10000
E = 320000
D = 128
DE = 16
H = 128
L = 2
EPS = 1e-5

NC = 2
NS = 16
NW = NC * NS
CH = 128
NCP = 2560
EP = NCP * CH
CPW = NCP // NW
NP = CPW // 2

NACC = N + 8
NT = N // CH
NTAIL = N - NT * CH
NTAILZ = NACC - NT * CH

BN_BLK = 1000
BE = 2560


@functools.cache
def _mesh():
    return plsc.VectorSubcoreMesh(core_axis_name="c", subcore_axis_name="s",
                                  num_cores=NC, num_subcores=NS)


def _fold_bn(W, b, bn):
    s = bn["g"] * jax.lax.rsqrt(bn["v"] + EPS)
    return W * s[None, :], b * s + (bn["b"] - bn["m"] * s)


def _zero_rows(buf, ncols, value=0.0):
    v16 = jnp.full((16,), value, jnp.float32)

    def row(i, carry):
        for k in range(ncols // 16):
            buf[i, pl.ds(k * 16, 16)] = v16
        return carry

    lax.fori_loop(0, CH, row, 0)


def _gather_body(pi_hbm, pj_hbm, dst_hbm, src_hbm, gi_hbm, gj_hbm,
                 di_v, si_v, bi_v, bj_v, sem_i, sem_j):
    cid = lax.axis_index("c")
    sid = lax.axis_index("s")
    wid = sid * NC + cid
    cbase = wid * CPW

    def step(k, carry):
        r = cbase + k
        pltpu.sync_copy(dst_hbm.at[pl.ds(r * CH, CH)], di_v)
        pltpu.sync_copy(src_hbm.at[pl.ds(r * CH, CH)], si_v)
        cp_i = pltpu.async_copy(pi_hbm.at[di_v], bi_v, sem_i)
        cp_j = pltpu.async_copy(pj_hbm.at[si_v], bj_v, sem_j)
        cp_i.wait()
        cp_j.wait()
        pltpu.sync_copy(bi_v, gi_hbm.at[pl.ds(r * CH, CH)])
        pltpu.sync_copy(bj_v, gj_hbm.at[pl.ds(r * CH, CH)])
        return carry

    lax.fori_loop(0, CPW, step, 0)


@functools.cache
def _sc_gather():
    return pl.kernel(
        _gather_body,
        out_type=[jax.ShapeDtypeStruct((EP, D), jnp.float32),
                  jax.ShapeDtypeStruct((EP, D), jnp.float32)],
        mesh=_mesh(),
        scratch_types=[
            pltpu.VMEM((CH,), jnp.int32),
            pltpu.VMEM((CH,), jnp.int32),
            pltpu.VMEM((CH, D), jnp.float32),
            pltpu.VMEM((CH, D), jnp.float32),
            pltpu.SemaphoreType.DMA,
            pltpu.SemaphoreType.DMA,
        ],
    )


def _scatter_body(msg_hbm, dst_hbm, agg_hbm, acc_sh, id2_v, m0, m1,
                  lsem0, lsem1, ssem0, ssem1, ones):
    cid = lax.axis_index("c")
    sid = lax.axis_index("s")
    wid = sid * NC + cid
    cbase = wid * CPW
    ntiles = (NT - sid + NS - 1) // NS

    _zero_rows(m0, D)

    def zchunk(i, carry):
        c = (sid + i * NS) * CH
        pltpu.sync_copy(m0, acc_sh.at[pl.ds(c, CH)])
        return carry

    lax.fori_loop(0, ntiles, zchunk, 0)

    @pl.when(sid == 0)
    def _ztail():
        pltpu.sync_copy(m0.at[pl.ds(0, NTAILZ)], acc_sh.at[pl.ds(NT * CH, NTAILZ)])

    pltpu.sync_copy(dst_hbm.at[pl.ds(cbase, CPW)], id2_v)
    if ones:
        _zero_rows(m0, D, value=1.0)
    plsc.subcore_barrier()

    def fire_load(k, m, sem):
        pltpu.async_copy(msg_hbm.at[pl.ds((cbase + k) * CH, CH)], m, sem)

    def wait_load(m, sem):
        pltpu.make_async_copy(msg_hbm.at[pl.ds(0, CH)], m, sem).wait()

    def fire_sadd(k, m, sem):
        pltpu.async_copy(m, acc_sh.at[id2_v.at[k]], sem, add=True)

    def wait_sadd(m, sem):
        pltpu.make_async_copy(m, acc_sh.at[id2_v.at[0]], sem).wait()

    if ones:
        def pair(i, carry):
            k0 = 2 * i
            fire_sadd(k0, m0, ssem0)
            fire_sadd(k0 + 1, m0, ssem1)
            wait_sadd(m0, ssem0)
            wait_sadd(m0, ssem1)
            return carry

        lax.fori_loop(0, NP, pair, 0)
    else:
        fire_load(0, m0, lsem0)

        def pair(i, carry):
            k0 = 2 * i
            k1 = k0 + 1

            @pl.when(i > 0)
            def _():
                wait_sadd(m1, ssem1)

            fire_load(k1, m1, lsem1)
            wait_load(m0, lsem0)
            fire_sadd(k0, m0, ssem0)
            wait_load(m1, lsem1)
            fire_sadd(k1, m1, ssem1)
            wait_sadd(m0, ssem0)

            @pl.when(i < NP - 1)
            def _():
                fire_load(k0 + 2, m0, lsem0)

            return carry

        lax.fori_loop(0, NP, pair, 0)
        wait_sadd(m1, ssem1)
    plsc.subcore_barrier()

    def wchunk(i, carry):
        c = (sid + i * NS) * CH
        pltpu.sync_copy(acc_sh.at[pl.ds(c, CH)], m0)
        pltpu.sync_copy(m0, agg_hbm.at[cid, pl.ds(c, CH)])
        return carry

    lax.fori_loop(0, ntiles, wchunk, 0)

    @pl.when(sid == 0)
    def _wtail():
        pltpu.sync_copy(acc_sh.at[pl.ds(NT * CH, NTAIL)], m0.at[pl.ds(0, NTAIL)])
        pltpu.sync_copy(m0.at[pl.ds(0, NTAIL)], agg_hbm.at[cid, pl.ds(NT * CH, NTAIL)])


def _cnt_body(dst_hbm, cnt_hbm, acc_sh, id2_v, m0, m1, ls0, ls1, ss0, ss1):
    _scatter_body(None, dst_hbm, cnt_hbm, acc_sh, id2_v, m0, m1,
                  ls0, ls1, ss0, ss1, ones=True)


def _msg_body(msg_hbm, dst_hbm, agg_hbm, acc_sh, id2_v, m0, m1, ls0, ls1, ss0, ss1):
    _scatter_body(msg_hbm, dst_hbm, agg_hbm, acc_sh, id2_v, m0, m1,
                  ls0, ls1, ss0, ss1, ones=False)


_SC_SCRATCH = [
    pltpu.VMEM_SHARED((NACC, D), jnp.float32),
    pltpu.VMEM((CPW, CH), jnp.int32),
    pltpu.VMEM((CH, D), jnp.float32),
    pltpu.VMEM((CH, D), jnp.float32),
    pltpu.SemaphoreType.DMA,
    pltpu.SemaphoreType.DMA,
    pltpu.SemaphoreType.DMA,
    pltpu.SemaphoreType.DMA,
]


@functools.cache
def _sc_scatter():
    return pl.kernel(
        _msg_body,
        out_type=[jax.ShapeDtypeStruct((NC, N, D), jnp.float32)],
        mesh=_mesh(),
        scratch_types=_SC_SCRATCH,
    )


@functools.cache
def _sc_cnt():
    return pl.kernel(
        _cnt_body,
        out_type=[jax.ShapeDtypeStruct((NC, N, D), jnp.float32)],
        mesh=_mesh(),
        scratch_types=_SC_SCRATCH,
    )


def _node_proj_body(x_ref, wi_ref, wj_ref, pi_ref, pj_ref):
    x = x_ref[...]
    pi_ref[...] = jnp.dot(x, wi_ref[...], preferred_element_type=jnp.float32)
    pj_ref[...] = jnp.dot(x, wj_ref[...], preferred_element_type=jnp.float32)


def _node_proj(x, Wi, Wj):
    return pl.pallas_call(
        _node_proj_body,
        grid=(N // BN_BLK,),
        in_specs=[pl.BlockSpec((BN_BLK, D), lambda i: (i, 0)),
                  pl.BlockSpec((D, H), lambda i: (0, 0)),
                  pl.BlockSpec((D, H), lambda i: (0, 0))],
        out_specs=[pl.BlockSpec((BN_BLK, H), lambda i: (i, 0)),
                   pl.BlockSpec((BN_BLK, H), lambda i: (i, 0))],
        out_shape=[jax.ShapeDtypeStruct((N, H), jnp.float32),
                   jax.ShapeDtypeStruct((N, H), jnp.float32)],
    )(x, Wi, Wj)


def _edge_mlp_body(gi_ref, gj_ref, e_ref, w1e_ref, b1_ref,
                   w2ij_ref, b2ij_ref, w2e_ref, b2e_ref, msg_ref, ne_ref):
    h = (gi_ref[...] + gj_ref[...]
         + jnp.dot(e_ref[...], w1e_ref[...], preferred_element_type=jnp.float32)
         + b1_ref[...])
    h = jnp.maximum(h, 0.0).astype(jnp.bfloat16)
    zij = jnp.dot(h, w2ij_ref[...], preferred_element_type=jnp.float32) + b2ij_ref[...]
    zij = jnp.maximum(zij, 0.0)
    msg_ref[...] = zij[:, :H] + zij[:, H:]
    ze = jnp.dot(h, w2e_ref[...], preferred_element_type=jnp.float32) + b2e_ref[...]
    ne_ref[...] = jnp.maximum(ze, 0.0)


def _edge_mlp(Gi, Gj, e, W1e, b1, W2ij, b2ij, W2e, b2e):
    return pl.pallas_call(
        _edge_mlp_body,
        grid=(EP // BE,),
        in_specs=[pl.BlockSpec((BE, H), lambda i: (i, 0)),
                  pl.BlockSpec((BE, H), lambda i: (i, 0)),
                  pl.BlockSpec((BE, DE), lambda i: (i, 0)),
                  pl.BlockSpec((DE, H), lambda i: (0, 0)),
                  pl.BlockSpec((1, H), lambda i: (0, 0)),
                  pl.BlockSpec((H, 2 * H), lambda i: (0, 0)),
                  pl.BlockSpec((1, 2 * H), lambda i: (0, 0)),
                  pl.BlockSpec((H, DE), lambda i: (0, 0)),
                  pl.BlockSpec((1, DE), lambda i: (0, 0))],
        out_specs=[pl.BlockSpec((BE, H), lambda i: (i, 0)),
                   pl.BlockSpec((BE, DE), lambda i: (i, 0))],
        out_shape=[jax.ShapeDtypeStruct((EP, H), jnp.float32),
                   jax.ShapeDtypeStruct((EP, DE), jnp.float32)],
    )(Gi, Gj, e, W1e, b1, W2ij, b2ij, W2e, b2e)


def _node_epi_body(x_ref, agg_ref, cnt_ref, v1_ref, c1_ref, v2_ref, c2_ref,
                   o_ref, *, final_relu):
    a = agg_ref[0] + agg_ref[1]
    cnt = jnp.maximum(cnt_ref[0, :, :1] + cnt_ref[1, :, :1], 1.0)
    gcn = a / cnt
    t = jnp.dot(gcn, v1_ref[...], preferred_element_type=jnp.float32) + c1_ref[...]
    t = jnp.maximum(t, 0.0)
    out = x_ref[...] + jnp.dot(t, v2_ref[...], preferred_element_type=jnp.float32) + c2_ref[...]
    if final_relu:
        out = jnp.maximum(out, 0.0)
    o_ref[...] = out


def _node_epi(x, agg2, cnt2, V1, c1, V2, c2, final_relu):
    return pl.pallas_call(
        functools.partial(_node_epi_body, final_relu=final_relu),
        grid=(N // BN_BLK,),
        in_specs=[pl.BlockSpec((BN_BLK, D), lambda i: (i, 0)),
                  pl.BlockSpec((NC, BN_BLK, D), lambda i: (0, i, 0)),
                  pl.BlockSpec((NC, BN_BLK, D), lambda i: (0, i, 0)),
                  pl.BlockSpec((H, H), lambda i: (0, 0)),
                  pl.BlockSpec((1, H), lambda i: (0, 0)),
                  pl.BlockSpec((H, D), lambda i: (0, 0)),
                  pl.BlockSpec((1, D), lambda i: (0, 0))],
        out_specs=pl.BlockSpec((BN_BLK, D), lambda i: (i, 0)),
        out_shape=jax.ShapeDtypeStruct((N, D), jnp.float32),
    )(x, agg2, cnt2, V1, c1, V2, c2)


def kernel(node_feature, edge_feature, edges_indices, params):
    x = node_feature
    src = edges_indices[0]
    dst = edges_indices[1]
    pad = EP - E
    zpad = jnp.zeros((pad,), jnp.int32)
    dst_g = jnp.concatenate([dst, zpad]).reshape(NCP, CH)
    src_g = jnp.concatenate([src, zpad]).reshape(NCP, CH)
    dst_s = jnp.concatenate([dst, jnp.full((pad,), N, jnp.int32)]).reshape(NCP, CH)
    e = jnp.concatenate([edge_feature, jnp.zeros((pad, DE), jnp.float32)], axis=0)

    cnt2 = _sc_cnt()(dst_s)[0]
    for l in range(L):
        p = params[l]
        W1, b1 = _fold_bn(p["nn1"]["w1"], p["nn1"]["b1"], p["nn1"]["bn1"])
        W2, b2 = _fold_bn(p["nn1"]["w2"], p["nn1"]["b2"], p["nn1"]["bn2"])
        W1i, W1e, W1j = W1[:D], W1[D:D + DE], W1[D + DE:]
        W2ij = jnp.concatenate([W2[:, :H], W2[:, H + DE:]], axis=1)
        b2ij = jnp.concatenate([b2[:H], b2[H + DE:]])[None, :]
        W2e = W2[:, H:H + DE]
        b2e = b2[H:H + DE][None, :]
        V1, c1 = _fold_bn(p["nn2"]["w1"], p["nn2"]["b1"], p["nn2"]["bn"])
        V2, c2 = p["nn2"]["w2"], p["nn2"]["b2"]

        Pi, Pj = _node_proj(x, W1i, W1j)
        Gi, Gj = _sc_gather()(Pi, Pj, dst_g.reshape(EP), src_g.reshape(EP))
        msg, ne = _edge_mlp(Gi, Gj, e, W1e, b1[None, :],
                            W2ij.astype(jnp.bfloat16), b2ij,
                            W2e.astype(jnp.bfloat16), b2e)
        (agg2,) = _sc_scatter()(msg, dst_s)
        x = _node_epi(x, agg2, cnt2, V1, c1[None, :], V2, c2[None, :],
                      final_relu=(l < L - 1))
        e = ne
    return (x, e[:E])


# speedup vs baseline: 1.5384x; 1.5384x over previous
import functools

import jax
import jax.numpy as jnp
from jax import lax
from jax.experimental import pallas as pl
from jax.experimental.pallas import tpu as pltpu
from jax.experimental.pallas import tpu_sc as plsc

N = 10000
E = 320000
D = 128
DE = 16
H = 128
L = 2
EPS = 1e-5

NC = 2
NS = 16
NW = NC * NS
CH = 128
NCP = 2560
EP = NCP * CH
CPW = NCP // NW
NP = CPW // 2

NACC = N + 8
NT = N // CH
NTAIL = N - NT * CH
NTAILZ = NACC - NT * CH

BN_BLK = 1000
BE = 2560


@functools.cache
def _mesh():
    return plsc.VectorSubcoreMesh(core_axis_name="c", subcore_axis_name="s",
                                  num_cores=NC, num_subcores=NS)


def _fold_bn(W, b, bn):
    s = bn["g"] * jax.lax.rsqrt(bn["v"] + EPS)
    return W * s[None, :], b * s + (bn["b"] - bn["m"] * s)


def _zero_rows(buf, ncols, value=0.0):
    v16 = jnp.full((16,), value, jnp.float32)

    def row(i, carry):
        for k in range(ncols // 16):
            buf[i, pl.ds(k * 16, 16)] = v16
        return carry

    lax.fori_loop(0, CH, row, 0)


def _gather_body(pi_hbm, pj_hbm, dst_hbm, src_hbm, gi_hbm, gj_hbm,
                 di_v, si_v, bi_v, bj_v, sem_i, sem_j):
    cid = lax.axis_index("c")
    sid = lax.axis_index("s")
    wid = sid * NC + cid
    cbase = wid * CPW

    def step(k, carry):
        r = cbase + k
        pltpu.sync_copy(dst_hbm.at[pl.ds(r * CH, CH)], di_v)
        pltpu.sync_copy(src_hbm.at[pl.ds(r * CH, CH)], si_v)
        cp_i = pltpu.async_copy(pi_hbm.at[di_v], bi_v, sem_i)
        cp_j = pltpu.async_copy(pj_hbm.at[si_v], bj_v, sem_j)
        cp_i.wait()
        cp_j.wait()
        pltpu.sync_copy(bi_v, gi_hbm.at[pl.ds(r * CH, CH)])
        pltpu.sync_copy(bj_v, gj_hbm.at[pl.ds(r * CH, CH)])
        return carry

    lax.fori_loop(0, CPW, step, 0)


@functools.cache
def _sc_gather():
    return pl.kernel(
        _gather_body,
        out_type=[jax.ShapeDtypeStruct((EP, D), jnp.float32),
                  jax.ShapeDtypeStruct((EP, D), jnp.float32)],
        mesh=_mesh(),
        scratch_types=[
            pltpu.VMEM((CH,), jnp.int32),
            pltpu.VMEM((CH,), jnp.int32),
            pltpu.VMEM((CH, D), jnp.float32),
            pltpu.VMEM((CH, D), jnp.float32),
            pltpu.SemaphoreType.DMA,
            pltpu.SemaphoreType.DMA,
        ],
    )


def _scatter_body(msg_hbm, dst_hbm, agg_hbm, acc_sh, id2_v, m0, m1,
                  lsem0, lsem1, ssem0, ssem1, ones):
    cid = lax.axis_index("c")
    sid = lax.axis_index("s")
    wid = sid * NC + cid
    cbase = wid * CPW
    ntiles = (NT - sid + NS - 1) // NS

    _zero_rows(m0, D)

    def zchunk(i, carry):
        c = (sid + i * NS) * CH
        pltpu.sync_copy(m0, acc_sh.at[pl.ds(c, CH)])
        return carry

    lax.fori_loop(0, ntiles, zchunk, 0)

    @pl.when(sid == 0)
    def _ztail():
        pltpu.sync_copy(m0.at[pl.ds(0, NTAILZ)], acc_sh.at[pl.ds(NT * CH, NTAILZ)])

    pltpu.sync_copy(dst_hbm.at[pl.ds(cbase, CPW)], id2_v)
    if ones:
        _zero_rows(m0, D, value=1.0)
    plsc.subcore_barrier()

    def fire_load(k, m, sem):
        pltpu.async_copy(msg_hbm.at[pl.ds((cbase + k) * CH, CH)], m, sem)

    def wait_load(m, sem):
        pltpu.make_async_copy(msg_hbm.at[pl.ds(0, CH)], m, sem).wait()

    def fire_sadd(k, m, sem):
        pltpu.async_copy(m, acc_sh.at[id2_v.at[k]], sem, add=True)

    def wait_sadd(m, sem):
        pltpu.make_async_copy(m, acc_sh.at[id2_v.at[0]], sem).wait()

    if ones:
        def pair(i, carry):
            k0 = 2 * i
            fire_sadd(k0, m0, ssem0)
            fire_sadd(k0 + 1, m0, ssem1)
            wait_sadd(m0, ssem0)
            wait_sadd(m0, ssem1)
            return carry

        lax.fori_loop(0, NP, pair, 0)
    else:
        fire_load(0, m0, lsem0)

        def pair(i, carry):
            k0 = 2 * i
            k1 = k0 + 1

            @pl.when(i > 0)
            def _():
                wait_sadd(m1, ssem1)

            fire_load(k1, m1, lsem1)
            wait_load(m0, lsem0)
            fire_sadd(k0, m0, ssem0)
            wait_load(m1, lsem1)
            fire_sadd(k1, m1, ssem1)
            wait_sadd(m0, ssem0)

            @pl.when(i < NP - 1)
            def _():
                fire_load(k0 + 2, m0, lsem0)

            return carry

        lax.fori_loop(0, NP, pair, 0)
        wait_sadd(m1, ssem1)
    plsc.subcore_barrier()

    def wchunk(i, carry):
        c = (sid + i * NS) * CH
        pltpu.sync_copy(acc_sh.at[pl.ds(c, CH)], m0)
        pltpu.sync_copy(m0, agg_hbm.at[cid, pl.ds(c, CH)])
        return carry

    lax.fori_loop(0, ntiles, wchunk, 0)

    @pl.when(sid == 0)
    def _wtail():
        pltpu.sync_copy(acc_sh.at[pl.ds(NT * CH, NTAIL)], m0.at[pl.ds(0, NTAIL)])
        pltpu.sync_copy(m0.at[pl.ds(0, NTAIL)], agg_hbm.at[cid, pl.ds(NT * CH, NTAIL)])


def _cnt_body(dst_hbm, cnt_hbm, acc_sh, id2_v, m0, m1, ls0, ls1, ss0, ss1):
    _scatter_body(None, dst_hbm, cnt_hbm, acc_sh, id2_v, m0, m1,
                  ls0, ls1, ss0, ss1, ones=True)


def _msg_body(msg_hbm, dst_hbm, agg_hbm, acc_sh, id2_v, m0, m1, ls0, ls1, ss0, ss1):
    _scatter_body(msg_hbm, dst_hbm, agg_hbm, acc_sh, id2_v, m0, m1,
                  ls0, ls1, ss0, ss1, ones=False)


_SC_SCRATCH = [
    pltpu.VMEM_SHARED((NACC, D), jnp.float32),
    pltpu.VMEM((CPW, CH), jnp.int32),
    pltpu.VMEM((CH, D), jnp.float32),
    pltpu.VMEM((CH, D), jnp.float32),
    pltpu.SemaphoreType.DMA,
    pltpu.SemaphoreType.DMA,
    pltpu.SemaphoreType.DMA,
    pltpu.SemaphoreType.DMA,
]


@functools.cache
def _sc_scatter():
    return pl.kernel(
        _msg_body,
        out_type=[jax.ShapeDtypeStruct((NC, N, D), jnp.float32)],
        mesh=_mesh(),
        scratch_types=_SC_SCRATCH,
    )


@functools.cache
def _sc_cnt():
    return pl.kernel(
        _cnt_body,
        out_type=[jax.ShapeDtypeStruct((NC, N, D), jnp.float32)],
        mesh=_mesh(),
        scratch_types=_SC_SCRATCH,
    )


def _node_proj_body(x_ref, wi_ref, wj_ref, pi_ref, pj_ref):
    x = x_ref[...]
    pi_ref[...] = jnp.dot(x, wi_ref[...], preferred_element_type=jnp.float32)
    pj_ref[...] = jnp.dot(x, wj_ref[...], preferred_element_type=jnp.float32)


def _node_proj(x, Wi, Wj):
    return pl.pallas_call(
        _node_proj_body,
        grid=(N // BN_BLK,),
        in_specs=[pl.BlockSpec((BN_BLK, D), lambda i: (i, 0)),
                  pl.BlockSpec((D, H), lambda i: (0, 0)),
                  pl.BlockSpec((D, H), lambda i: (0, 0))],
        out_specs=[pl.BlockSpec((BN_BLK, H), lambda i: (i, 0)),
                   pl.BlockSpec((BN_BLK, H), lambda i: (i, 0))],
        out_shape=[jax.ShapeDtypeStruct((N, H), jnp.float32),
                   jax.ShapeDtypeStruct((N, H), jnp.float32)],
    )(x, Wi, Wj)


def _edge_mlp_body(gi_ref, gj_ref, e_ref, w1e_ref, b1_ref,
                   w2ij_ref, b2ij_ref, w2e_ref, b2e_ref, msg_ref, ne_ref):
    h = (gi_ref[...] + gj_ref[...]
         + jnp.dot(e_ref[...], w1e_ref[...], preferred_element_type=jnp.float32)
         + b1_ref[...])
    h = jnp.maximum(h, 0.0).astype(jnp.bfloat16)
    zij = jnp.dot(h, w2ij_ref[...], preferred_element_type=jnp.float32) + b2ij_ref[...]
    zij = jnp.maximum(zij, 0.0)
    msg_ref[...] = zij[:, :H] + zij[:, H:]
    ze = jnp.dot(h, w2e_ref[...], preferred_element_type=jnp.float32) + b2e_ref[...]
    ne_ref[...] = jnp.maximum(ze, 0.0)


def _edge_mlp(Gi, Gj, e, W1e, b1, W2ij, b2ij, W2e, b2e):
    return pl.pallas_call(
        _edge_mlp_body,
        grid=(EP // BE,),
        in_specs=[pl.BlockSpec((BE, H), lambda i: (i, 0)),
                  pl.BlockSpec((BE, H), lambda i: (i, 0)),
                  pl.BlockSpec((BE, DE), lambda i: (i, 0)),
                  pl.BlockSpec((DE, H), lambda i: (0, 0)),
                  pl.BlockSpec((1, H), lambda i: (0, 0)),
                  pl.BlockSpec((H, 2 * H), lambda i: (0, 0)),
                  pl.BlockSpec((1, 2 * H), lambda i: (0, 0)),
                  pl.BlockSpec((H, DE), lambda i: (0, 0)),
                  pl.BlockSpec((1, DE), lambda i: (0, 0))],
        out_specs=[pl.BlockSpec((BE, H), lambda i: (i, 0)),
                   pl.BlockSpec((BE, DE), lambda i: (i, 0))],
        out_shape=[jax.ShapeDtypeStruct((EP, H), jnp.float32),
                   jax.ShapeDtypeStruct((EP, DE), jnp.float32)],
    )(Gi, Gj, e, W1e, b1, W2ij, b2ij, W2e, b2e)


def _node_epi_body(x_ref, agg_ref, cnt_ref, v1_ref, c1_ref, v2_ref, c2_ref,
                   o_ref, *, final_relu):
    a = agg_ref[0] + agg_ref[1]
    cnt = jnp.maximum(cnt_ref[0, :, :1] + cnt_ref[1, :, :1], 1.0)
    gcn = a / cnt
    t = jnp.dot(gcn, v1_ref[...], preferred_element_type=jnp.float32) + c1_ref[...]
    t = jnp.maximum(t, 0.0)
    out = x_ref[...] + jnp.dot(t, v2_ref[...], preferred_element_type=jnp.float32) + c2_ref[...]
    if final_relu:
        out = jnp.maximum(out, 0.0)
    o_ref[...] = out


def _node_epi(x, agg2, cnt2, V1, c1, V2, c2, final_relu):
    return pl.pallas_call(
        functools.partial(_node_epi_body, final_relu=final_relu),
        grid=(N // BN_BLK,),
        in_specs=[pl.BlockSpec((BN_BLK, D), lambda i: (i, 0)),
                  pl.BlockSpec((NC, BN_BLK, D), lambda i: (0, i, 0)),
                  pl.BlockSpec((NC, BN_BLK, D), lambda i: (0, i, 0)),
                  pl.BlockSpec((H, H), lambda i: (0, 0)),
                  pl.BlockSpec((1, H), lambda i: (0, 0)),
                  pl.BlockSpec((H, D), lambda i: (0, 0)),
                  pl.BlockSpec((1, D), lambda i: (0, 0))],
        out_specs=pl.BlockSpec((BN_BLK, D), lambda i: (i, 0)),
        out_shape=jax.ShapeDtypeStruct((N, D), jnp.float32),
    )(x, agg2, cnt2, V1, c1, V2, c2)


def kernel(node_feature, edge_feature, edges_indices, params):
    x = node_feature
    src = edges_indices[0]
    dst = edges_indices[1]
    pad = EP - E
    padidx = jnp.arange(pad, dtype=jnp.int32)
    dst_g = jnp.concatenate([dst, padidx % N]).reshape(NCP, CH)
    src_g = jnp.concatenate([src, padidx % N]).reshape(NCP, CH)
    dst_s = jnp.concatenate([dst, N + (padidx % 8)]).reshape(NCP, CH)
    e = jnp.concatenate([edge_feature, jnp.zeros((pad, DE), jnp.float32)], axis=0)

    cnt2 = _sc_cnt()(dst_s)[0]
    for l in range(L):
        p = params[l]
        W1, b1 = _fold_bn(p["nn1"]["w1"], p["nn1"]["b1"], p["nn1"]["bn1"])
        W2, b2 = _fold_bn(p["nn1"]["w2"], p["nn1"]["b2"], p["nn1"]["bn2"])
        W1i, W1e, W1j = W1[:D], W1[D:D + DE], W1[D + DE:]
        W2ij = jnp.concatenate([W2[:, :H], W2[:, H + DE:]], axis=1)
        b2ij = jnp.concatenate([b2[:H], b2[H + DE:]])[None, :]
        W2e = W2[:, H:H + DE]
        b2e = b2[H:H + DE][None, :]
        V1, c1 = _fold_bn(p["nn2"]["w1"], p["nn2"]["b1"], p["nn2"]["bn"])
        V2, c2 = p["nn2"]["w2"], p["nn2"]["b2"]

        Pi, Pj = _node_proj(x, W1i, W1j)
        Gi, Gj = _sc_gather()(Pi, Pj, dst_g.reshape(EP), src_g.reshape(EP))
        msg, ne = _edge_mlp(Gi, Gj, e, W1e, b1[None, :],
                            W2ij.astype(jnp.bfloat16), b2ij,
                            W2e.astype(jnp.bfloat16), b2e)
        (agg2,) = _sc_scatter()(msg, dst_s)
        x = _node_epi(x, agg2, cnt2, V1, c1[None, :], V2, c2[None, :],
                      final_relu=(l < L - 1))
        e = ne
    return (x, e[:E])


# speedup vs baseline: 1.7161x; 1.1155x over previous
import functools

import jax
import jax.numpy as jnp
from jax import lax
from jax.experimental import pallas as pl
from jax.experimental.pallas import tpu as pltpu
from jax.experimental.pallas import tpu_sc as plsc

N = 10000
E = 320000
D = 128
DE = 16
H = 128
L = 2
EPS = 1e-5

NC = 2
NS = 16
NW = NC * NS
CH = 128
NCP = 2560
EP = NCP * CH
CPW = NCP // NW
NP = CPW // 2

NACC = N + 8
NT = N // CH
NTAIL = N - NT * CH
NTAILZ = NACC - NT * CH

BN_BLK = 1000
BE = 2560


@functools.cache
def _mesh():
    return plsc.VectorSubcoreMesh(core_axis_name="c", subcore_axis_name="s",
                                  num_cores=NC, num_subcores=NS)


def _fold_bn(W, b, bn):
    s = bn["g"] * jax.lax.rsqrt(bn["v"] + EPS)
    return W * s[None, :], b * s + (bn["b"] - bn["m"] * s)


def _zero_rows(buf, ncols, value=0.0):
    v16 = jnp.full((16,), value, jnp.float32)

    def row(i, carry):
        for k in range(ncols // 16):
            buf[i, pl.ds(k * 16, 16)] = v16
        return carry

    lax.fori_loop(0, CH, row, 0)


def _gather_body(pi_hbm, pj_hbm, dst_hbm, src_hbm, gi_hbm, gj_hbm,
                 id_v, is_v, bi0, bj0, bi1, bj1, gsem0, gsem1, wsem0, wsem1):
    cid = lax.axis_index("c")
    sid = lax.axis_index("s")
    wid = sid * NC + cid
    cbase = wid * CPW

    pltpu.sync_copy(dst_hbm.at[pl.ds(cbase, CPW)], id_v)
    pltpu.sync_copy(src_hbm.at[pl.ds(cbase, CPW)], is_v)

    def fire_gather(k, bi, bj, sem):
        pltpu.async_copy(pi_hbm.at[id_v.at[k]], bi, sem)
        pltpu.async_copy(pj_hbm.at[is_v.at[k]], bj, sem)

    def wait_gather(bi, bj, sem):
        pltpu.make_async_copy(pi_hbm.at[id_v.at[0]], bi, sem).wait()
        pltpu.make_async_copy(pj_hbm.at[is_v.at[0]], bj, sem).wait()

    def fire_write(k, bi, bj, sem):
        pltpu.async_copy(bi, gi_hbm.at[pl.ds((cbase + k) * CH, CH)], sem)
        pltpu.async_copy(bj, gj_hbm.at[pl.ds((cbase + k) * CH, CH)], sem)

    def wait_write(bi, bj, sem):
        pltpu.make_async_copy(bi, gi_hbm.at[pl.ds(0, CH)], sem).wait()
        pltpu.make_async_copy(bj, gj_hbm.at[pl.ds(0, CH)], sem).wait()

    fire_gather(0, bi0, bj0, gsem0)

    def pair(i, carry):
        k0 = 2 * i
        k1 = k0 + 1

        @pl.when(i > 0)
        def _():
            wait_write(bi1, bj1, wsem1)

        fire_gather(k1, bi1, bj1, gsem1)
        wait_gather(bi0, bj0, gsem0)
        fire_write(k0, bi0, bj0, wsem0)
        wait_gather(bi1, bj1, gsem1)
        fire_write(k1, bi1, bj1, wsem1)
        wait_write(bi0, bj0, wsem0)

        @pl.when(i < NP - 1)
        def _():
            fire_gather(k0 + 2, bi0, bj0, gsem0)

        return carry

    lax.fori_loop(0, NP, pair, 0)
    wait_write(bi1, bj1, wsem1)


@functools.cache
def _sc_gather():
    return pl.kernel(
        _gather_body,
        out_type=[jax.ShapeDtypeStruct((EP, D), jnp.float32),
                  jax.ShapeDtypeStruct((EP, D), jnp.float32)],
        mesh=_mesh(),
        scratch_types=[
            pltpu.VMEM((CPW, CH), jnp.int32),
            pltpu.VMEM((CPW, CH), jnp.int32),
            pltpu.VMEM((CH, D), jnp.float32),
            pltpu.VMEM((CH, D), jnp.float32),
            pltpu.VMEM((CH, D), jnp.float32),
            pltpu.VMEM((CH, D), jnp.float32),
            pltpu.SemaphoreType.DMA,
            pltpu.SemaphoreType.DMA,
            pltpu.SemaphoreType.DMA,
            pltpu.SemaphoreType.DMA,
        ],
    )


def _scatter_body(msg_hbm, dst_hbm, agg_hbm, acc_sh, id2_v, m0, m1,
                  lsem0, lsem1, ssem0, ssem1, ones):
    cid = lax.axis_index("c")
    sid = lax.axis_index("s")
    wid = sid * NC + cid
    cbase = wid * CPW
    ntiles = (NT - sid + NS - 1) // NS

    _zero_rows(m0, D)

    def zchunk(i, carry):
        c = (sid + i * NS) * CH
        pltpu.sync_copy(m0, acc_sh.at[pl.ds(c, CH)])
        return carry

    lax.fori_loop(0, ntiles, zchunk, 0)

    @pl.when(sid == 0)
    def _ztail():
        pltpu.sync_copy(m0.at[pl.ds(0, NTAILZ)], acc_sh.at[pl.ds(NT * CH, NTAILZ)])

    pltpu.sync_copy(dst_hbm.at[pl.ds(cbase, CPW)], id2_v)
    if ones:
        _zero_rows(m0, D, value=1.0)
    plsc.subcore_barrier()

    def fire_load(k, m, sem):
        pltpu.async_copy(msg_hbm.at[pl.ds((cbase + k) * CH, CH)], m, sem)

    def wait_load(m, sem):
        pltpu.make_async_copy(msg_hbm.at[pl.ds(0, CH)], m, sem).wait()

    def fire_sadd(k, m, sem):
        pltpu.async_copy(m, acc_sh.at[id2_v.at[k]], sem, add=True)

    def wait_sadd(m, sem):
        pltpu.make_async_copy(m, acc_sh.at[id2_v.at[0]], sem).wait()

    if ones:
        def pair(i, carry):
            k0 = 2 * i
            fire_sadd(k0, m0, ssem0)
            fire_sadd(k0 + 1, m0, ssem1)
            wait_sadd(m0, ssem0)
            wait_sadd(m0, ssem1)
            return carry

        lax.fori_loop(0, NP, pair, 0)
    else:
        fire_load(0, m0, lsem0)

        def pair(i, carry):
            k0 = 2 * i
            k1 = k0 + 1

            @pl.when(i > 0)
            def _():
                wait_sadd(m1, ssem1)

            fire_load(k1, m1, lsem1)
            wait_load(m0, lsem0)
            fire_sadd(k0, m0, ssem0)
            wait_load(m1, lsem1)
            fire_sadd(k1, m1, ssem1)
            wait_sadd(m0, ssem0)

            @pl.when(i < NP - 1)
            def _():
                fire_load(k0 + 2, m0, lsem0)

            return carry

        lax.fori_loop(0, NP, pair, 0)
        wait_sadd(m1, ssem1)
    plsc.subcore_barrier()

    def wchunk(i, carry):
        c = (sid + i * NS) * CH
        pltpu.sync_copy(acc_sh.at[pl.ds(c, CH)], m0)
        pltpu.sync_copy(m0, agg_hbm.at[cid, pl.ds(c, CH)])
        return carry

    lax.fori_loop(0, ntiles, wchunk, 0)

    @pl.when(sid == 0)
    def _wtail():
        pltpu.sync_copy(acc_sh.at[pl.ds(NT * CH, NTAIL)], m0.at[pl.ds(0, NTAIL)])
        pltpu.sync_copy(m0.at[pl.ds(0, NTAIL)], agg_hbm.at[cid, pl.ds(NT * CH, NTAIL)])


def _cnt_body(dst_hbm, cnt_hbm, acc_sh, id2_v, m0, m1, ls0, ls1, ss0, ss1):
    _scatter_body(None, dst_hbm, cnt_hbm, acc_sh, id2_v, m0, m1,
                  ls0, ls1, ss0, ss1, ones=True)


def _msg_body(msg_hbm, dst_hbm, agg_hbm, acc_sh, id2_v, m0, m1, ls0, ls1, ss0, ss1):
    _scatter_body(msg_hbm, dst_hbm, agg_hbm, acc_sh, id2_v, m0, m1,
                  ls0, ls1, ss0, ss1, ones=False)


_SC_SCRATCH = [
    pltpu.VMEM_SHARED((NACC, D), jnp.float32),
    pltpu.VMEM((CPW, CH), jnp.int32),
    pltpu.VMEM((CH, D), jnp.float32),
    pltpu.VMEM((CH, D), jnp.float32),
    pltpu.SemaphoreType.DMA,
    pltpu.SemaphoreType.DMA,
    pltpu.SemaphoreType.DMA,
    pltpu.SemaphoreType.DMA,
]


@functools.cache
def _sc_scatter():
    return pl.kernel(
        _msg_body,
        out_type=[jax.ShapeDtypeStruct((NC, N, D), jnp.float32)],
        mesh=_mesh(),
        scratch_types=_SC_SCRATCH,
    )


@functools.cache
def _sc_cnt():
    return pl.kernel(
        _cnt_body,
        out_type=[jax.ShapeDtypeStruct((NC, N, D), jnp.float32)],
        mesh=_mesh(),
        scratch_types=_SC_SCRATCH,
    )


def _node_proj_body(x_ref, wi_ref, wj_ref, pi_ref, pj_ref):
    x = x_ref[...]
    pi_ref[...] = jnp.dot(x, wi_ref[...], preferred_element_type=jnp.float32)
    pj_ref[...] = jnp.dot(x, wj_ref[...], preferred_element_type=jnp.float32)


def _node_proj(x, Wi, Wj):
    return pl.pallas_call(
        _node_proj_body,
        grid=(N // BN_BLK,),
        in_specs=[pl.BlockSpec((BN_BLK, D), lambda i: (i, 0)),
                  pl.BlockSpec((D, H), lambda i: (0, 0)),
                  pl.BlockSpec((D, H), lambda i: (0, 0))],
        out_specs=[pl.BlockSpec((BN_BLK, H), lambda i: (i, 0)),
                   pl.BlockSpec((BN_BLK, H), lambda i: (i, 0))],
        out_shape=[jax.ShapeDtypeStruct((N, H), jnp.float32),
                   jax.ShapeDtypeStruct((N, H), jnp.float32)],
    )(x, Wi, Wj)


def _edge_mlp_body(gi_ref, gj_ref, e_ref, w1e_ref, b1_ref,
                   w2ij_ref, b2ij_ref, w2e_ref, b2e_ref, msg_ref, ne_ref):
    h = (gi_ref[...] + gj_ref[...]
         + jnp.dot(e_ref[...], w1e_ref[...], preferred_element_type=jnp.float32)
         + b1_ref[...])
    h = jnp.maximum(h, 0.0).astype(jnp.bfloat16)
    zij = jnp.dot(h, w2ij_ref[...], preferred_element_type=jnp.float32) + b2ij_ref[...]
    zij = jnp.maximum(zij, 0.0)
    msg_ref[...] = zij[:, :H] + zij[:, H:]
    ze = jnp.dot(h, w2e_ref[...], preferred_element_type=jnp.float32) + b2e_ref[...]
    ne_ref[...] = jnp.maximum(ze, 0.0)


def _edge_mlp(Gi, Gj, e, W1e, b1, W2ij, b2ij, W2e, b2e):
    return pl.pallas_call(
        _edge_mlp_body,
        grid=(EP // BE,),
        in_specs=[pl.BlockSpec((BE, H), lambda i: (i, 0)),
                  pl.BlockSpec((BE, H), lambda i: (i, 0)),
                  pl.BlockSpec((BE, DE), lambda i: (i, 0)),
                  pl.BlockSpec((DE, H), lambda i: (0, 0)),
                  pl.BlockSpec((1, H), lambda i: (0, 0)),
                  pl.BlockSpec((H, 2 * H), lambda i: (0, 0)),
                  pl.BlockSpec((1, 2 * H), lambda i: (0, 0)),
                  pl.BlockSpec((H, DE), lambda i: (0, 0)),
                  pl.BlockSpec((1, DE), lambda i: (0, 0))],
        out_specs=[pl.BlockSpec((BE, H), lambda i: (i, 0)),
                   pl.BlockSpec((BE, DE), lambda i: (i, 0))],
        out_shape=[jax.ShapeDtypeStruct((EP, H), jnp.float32),
                   jax.ShapeDtypeStruct((EP, DE), jnp.float32)],
    )(Gi, Gj, e, W1e, b1, W2ij, b2ij, W2e, b2e)


def _node_epi_body(x_ref, agg_ref, cnt_ref, v1_ref, c1_ref, v2_ref, c2_ref,
                   o_ref, *, final_relu):
    a = agg_ref[0] + agg_ref[1]
    cnt = jnp.maximum(cnt_ref[0, :, :1] + cnt_ref[1, :, :1], 1.0)
    gcn = a / cnt
    t = jnp.dot(gcn, v1_ref[...], preferred_element_type=jnp.float32) + c1_ref[...]
    t = jnp.maximum(t, 0.0)
    out = x_ref[...] + jnp.dot(t, v2_ref[...], preferred_element_type=jnp.float32) + c2_ref[...]
    if final_relu:
        out = jnp.maximum(out, 0.0)
    o_ref[...] = out


def _node_epi(x, agg2, cnt2, V1, c1, V2, c2, final_relu):
    return pl.pallas_call(
        functools.partial(_node_epi_body, final_relu=final_relu),
        grid=(N // BN_BLK,),
        in_specs=[pl.BlockSpec((BN_BLK, D), lambda i: (i, 0)),
                  pl.BlockSpec((NC, BN_BLK, D), lambda i: (0, i, 0)),
                  pl.BlockSpec((NC, BN_BLK, D), lambda i: (0, i, 0)),
                  pl.BlockSpec((H, H), lambda i: (0, 0)),
                  pl.BlockSpec((1, H), lambda i: (0, 0)),
                  pl.BlockSpec((H, D), lambda i: (0, 0)),
                  pl.BlockSpec((1, D), lambda i: (0, 0))],
        out_specs=pl.BlockSpec((BN_BLK, D), lambda i: (i, 0)),
        out_shape=jax.ShapeDtypeStruct((N, D), jnp.float32),
    )(x, agg2, cnt2, V1, c1, V2, c2)


def kernel(node_feature, edge_feature, edges_indices, params):
    x = node_feature
    src = edges_indices[0]
    dst = edges_indices[1]
    pad = EP - E
    padidx = jnp.arange(pad, dtype=jnp.int32)
    dst_g = jnp.concatenate([dst, padidx % N]).reshape(NCP, CH)
    src_g = jnp.concatenate([src, padidx % N]).reshape(NCP, CH)
    dst_s = jnp.concatenate([dst, N + (padidx % 8)]).reshape(NCP, CH)
    e = jnp.concatenate([edge_feature, jnp.zeros((pad, DE), jnp.float32)], axis=0)

    cnt2 = _sc_cnt()(dst_s)[0]
    for l in range(L):
        p = params[l]
        W1, b1 = _fold_bn(p["nn1"]["w1"], p["nn1"]["b1"], p["nn1"]["bn1"])
        W2, b2 = _fold_bn(p["nn1"]["w2"], p["nn1"]["b2"], p["nn1"]["bn2"])
        W1i, W1e, W1j = W1[:D], W1[D:D + DE], W1[D + DE:]
        W2ij = jnp.concatenate([W2[:, :H], W2[:, H + DE:]], axis=1)
        b2ij = jnp.concatenate([b2[:H], b2[H + DE:]])[None, :]
        W2e = W2[:, H:H + DE]
        b2e = b2[H:H + DE][None, :]
        V1, c1 = _fold_bn(p["nn2"]["w1"], p["nn2"]["b1"], p["nn2"]["bn"])
        V2, c2 = p["nn2"]["w2"], p["nn2"]["b2"]

        Pi, Pj = _node_proj(x, W1i, W1j)
        Gi, Gj = _sc_gather()(Pi, Pj, dst_g, src_g)
        msg, ne = _edge_mlp(Gi, Gj, e, W1e, b1[None, :],
                            W2ij.astype(jnp.bfloat16), b2ij,
                            W2e.astype(jnp.bfloat16), b2e)
        (agg2,) = _sc_scatter()(msg, dst_s)
        x = _node_epi(x, agg2, cnt2, V1, c1[None, :], V2, c2[None, :],
                      final_relu=(l < L - 1))
        e = ne
    return (x, e[:E])
